# Initial kernel scaffold; baseline (speedup 1.0000x reference)
import jax
import jax.numpy as jnp
from jax.experimental import pallas as pl


def kernel(edge_index, edge_feat, node_feat, state_feat, params):
    raise NotImplementedError("write your pallas kernel here")


# speedup vs baseline: 1.6484x; 1.6484x over previous
import functools

import jax
import jax.numpy as jnp
from jax import lax
from jax.experimental import pallas as pl
from jax.experimental.pallas import tpu as pltpu
from jax.experimental.pallas import tpu_sc as plsc

N_NODES = 10000
N_EDGES = 320000
D = 32

NC, NS = 2, 16
NW = NC * NS
EPW = N_EDGES // NW
CH = 80
NCHUNK = EPW // CH
NPS = N_NODES // NS

BE = 2560
BN = 2000

_F32 = jnp.float32


def _sp2(x):
    return jnp.maximum(x, 0.0) + jnp.log1p(jnp.exp(-jnp.abs(x))) - 0.6931471805599453


def _dot(a, b):
    return jnp.dot(a, b, preferred_element_type=_F32)


def _sc_mesh():
    return plsc.VectorSubcoreMesh(core_axis_name="c", subcore_axis_name="s",
                                  num_cores=NC, num_subcores=NS)


_SC_PARAMS = pltpu.CompilerParams(use_tc_tiling_on_sc=False)


def _sc_gather2(vp, src, dst):
    @functools.partial(
        pl.kernel,
        out_type=(jax.ShapeDtypeStruct((N_EDGES, D), _F32),
                  jax.ShapeDtypeStruct((N_EDGES, D), _F32)),
        mesh=_sc_mesh(),
        compiler_params=_SC_PARAMS,
        scratch_types=[
            pltpu.VMEM((CH,), jnp.int32),
            pltpu.VMEM((CH, D), _F32),
            pltpu.SemaphoreType.DMA,
        ],
    )
    def k(vp_hbm, src_hbm, dst_hbm, os_hbm, od_hbm, idx_v, rows_v, sem):
        wid = lax.axis_index("s") * NC + lax.axis_index("c")
        base = wid * EPW

        def body(i, carry):
            off = base + i * CH
            pltpu.sync_copy(src_hbm.at[pl.ds(off, CH)], idx_v)
            pltpu.async_copy(vp_hbm.at[idx_v], rows_v, sem).wait()
            pltpu.sync_copy(rows_v, os_hbm.at[pl.ds(off, CH)])
            pltpu.sync_copy(dst_hbm.at[pl.ds(off, CH)], idx_v)
            pltpu.async_copy(vp_hbm.at[idx_v], rows_v, sem).wait()
            pltpu.sync_copy(rows_v, od_hbm.at[pl.ds(off, CH)])
            return carry

        lax.fori_loop(0, NCHUNK, body, 0, unroll=False)

    return k(vp, src, dst)


def _sc_scatter(e_new, dst, zrows, orows, with_counts):
    outs = [jax.ShapeDtypeStruct((NC, N_NODES, D), _F32)]
    scratch = [
        pltpu.VMEM_SHARED((N_NODES, D), _F32),
        pltpu.VMEM((CH,), jnp.int32),
        pltpu.VMEM((CH, D), _F32),
    ]
    if with_counts:
        outs.append(jax.ShapeDtypeStruct((NC, N_NODES, D), _F32))
        scratch += [pltpu.VMEM_SHARED((N_NODES, D), _F32),
                    pltpu.VMEM((CH, D), _F32)]

    @functools.partial(pl.kernel, out_type=tuple(outs), mesh=_sc_mesh(),
                       compiler_params=_SC_PARAMS, scratch_types=scratch)
    def k(enew_hbm, dst_hbm, z_hbm, o_hbm, *rest):
        if with_counts:
            (psum_hbm, cnt_hbm, shared, idx_v, rows_v, shared_cnt, ones_v) = rest
        else:
            (psum_hbm, shared, idx_v, rows_v) = rest
        cid = lax.axis_index("c")
        sid = lax.axis_index("s")
        wid = sid * NC + cid
        pltpu.sync_copy(z_hbm, shared.at[pl.ds(sid * NPS, NPS)])
        if with_counts:
            pltpu.sync_copy(z_hbm, shared_cnt.at[pl.ds(sid * NPS, NPS)])
            pltpu.sync_copy(o_hbm, ones_v)
        plsc.subcore_barrier()

        base = wid * EPW

        def body(i, carry):
            off = base + i * CH
            pltpu.sync_copy(dst_hbm.at[pl.ds(off, CH)], idx_v)
            pltpu.sync_copy(enew_hbm.at[pl.ds(off, CH)], rows_v)
            pltpu.sync_copy(rows_v, shared.at[idx_v], add=True)
            if with_counts:
                pltpu.sync_copy(ones_v, shared_cnt.at[idx_v], add=True)
            return carry

        lax.fori_loop(0, NCHUNK, body, 0, unroll=False)
        plsc.subcore_barrier()
        pltpu.sync_copy(shared.at[pl.ds(sid * NPS, NPS)],
                        psum_hbm.at[cid, pl.ds(sid * NPS, NPS)])
        if with_counts:
            pltpu.sync_copy(shared_cnt.at[pl.ds(sid * NPS, NPS)],
                            cnt_hbm.at[cid, pl.ds(sid * NPS, NPS)])

    return k(e_new, dst, zrows, orows)


def _full(shape):
    return pl.BlockSpec(shape, lambda i: (0,) * len(shape))


def _edge_kernel_body(has_pre, *refs):
    (x_ref, vs_ref, vd_ref, q1_ref, r1_ref, q2_ref, r2_ref,
     a_ref, b_ref, c_ref, ub_ref, w2_ref, b2_ref, w3_ref, b3_ref,
     enew_ref, eout_ref, esum_ref) = refs
    j = pl.program_id(0)

    x = x_ref[...]
    ep = _sp2(_dot(x, q1_ref[...]) + r1_ref[...])
    ep = _sp2(_dot(ep, q2_ref[...]) + r2_ref[...])
    e_res = ep if not has_pre else x

    h = (_dot(vs_ref[...], a_ref[...]) + _dot(vd_ref[...], b_ref[...])
         + _dot(ep, c_ref[...]) + ub_ref[...])
    h = _sp2(h)
    h = _sp2(_dot(h, w2_ref[...]) + b2_ref[...])
    en = _sp2(_dot(h, w3_ref[...]) + b3_ref[...])

    enew_ref[...] = en
    eout_ref[...] = en + e_res

    @pl.when(j == 0)
    def _():
        esum_ref[...] = jnp.zeros_like(esum_ref)

    esum_ref[...] += jnp.sum(en, axis=0, keepdims=True)


def _edge_block(x, vs, vd, pre_w, f_w, ubias, has_pre):
    xin = x.shape[1]
    (q1, r1), (q2, r2) = pre_w
    (w1, _b1), (w2, b2), (w3, b3) = f_w
    a, b, c = w1[0:32], w1[32:64], w1[64:96]
    grid = (N_EDGES // BE,)
    body = functools.partial(_edge_kernel_body, has_pre)
    return pl.pallas_call(
        body,
        grid=grid,
        in_specs=[
            pl.BlockSpec((BE, xin), lambda i: (i, 0)),
            pl.BlockSpec((BE, D), lambda i: (i, 0)),
            pl.BlockSpec((BE, D), lambda i: (i, 0)),
            _full(q1.shape), _full((1, r1.shape[0])),
            _full(q2.shape), _full((1, r2.shape[0])),
            _full(a.shape), _full(b.shape), _full(c.shape), _full(ubias.shape),
            _full(w2.shape), _full((1, b2.shape[0])),
            _full(w3.shape), _full((1, b3.shape[0])),
        ],
        out_specs=[
            pl.BlockSpec((BE, D), lambda i: (i, 0)),
            pl.BlockSpec((BE, D), lambda i: (i, 0)),
            pl.BlockSpec((1, D), lambda i: (0, 0)),
        ],
        out_shape=[
            jax.ShapeDtypeStruct((N_EDGES, D), _F32),
            jax.ShapeDtypeStruct((N_EDGES, D), _F32),
            jax.ShapeDtypeStruct((1, D), _F32),
        ],
    )(x, vs, vd, q1, r1.reshape(1, -1), q2, r2.reshape(1, -1),
      a, b, c, ubias, w2, b2.reshape(1, -1), w3, b3.reshape(1, -1))


def _node_encode_body(idx_ref, tab_ref, q1_ref, r1_ref, q2_ref, r2_ref, out_ref):
    idx = idx_ref[...]
    iota = lax.broadcasted_iota(jnp.int32, (BN, 96), 1)
    oh = (iota == idx).astype(_F32)
    v = _dot(oh, tab_ref[...])
    v = _sp2(_dot(v, q1_ref[...]) + r1_ref[...])
    out_ref[...] = _sp2(_dot(v, q2_ref[...]) + r2_ref[...])


def _node_encode(node_feat, table, enc_w):
    tab = jnp.pad(table, ((0, 96 - table.shape[0]), (0, 0)))
    idx2 = node_feat.reshape(N_NODES, 1).astype(jnp.int32)
    (q1, r1), (q2, r2) = enc_w
    return pl.pallas_call(
        _node_encode_body,
        grid=(N_NODES // BN,),
        in_specs=[
            pl.BlockSpec((BN, 1), lambda i: (i, 0)),
            _full(tab.shape),
            _full(q1.shape), _full((1, r1.shape[0])),
            _full(q2.shape), _full((1, r2.shape[0])),
        ],
        out_specs=pl.BlockSpec((BN, D), lambda i: (i, 0)),
        out_shape=jax.ShapeDtypeStruct((N_NODES, D), _F32),
    )(idx2, tab, q1, r1.reshape(1, -1), q2, r2.reshape(1, -1))


def _mlp2_body(x_ref, q1_ref, r1_ref, q2_ref, r2_ref, out_ref):
    x = x_ref[...]
    x = _sp2(_dot(x, q1_ref[...]) + r1_ref[...])
    out_ref[...] = _sp2(_dot(x, q2_ref[...]) + r2_ref[...])


def _pre_node(v, pre_w):
    (q1, r1), (q2, r2) = pre_w
    return pl.pallas_call(
        _mlp2_body,
        grid=(N_NODES // BN,),
        in_specs=[
            pl.BlockSpec((BN, D), lambda i: (i, 0)),
            _full(q1.shape), _full((1, r1.shape[0])),
            _full(q2.shape), _full((1, r2.shape[0])),
        ],
        out_specs=pl.BlockSpec((BN, D), lambda i: (i, 0)),
        out_shape=jax.ShapeDtypeStruct((N_NODES, D), _F32),
    )(v, q1, r1.reshape(1, -1), q2, r2.reshape(1, -1))


def _node_block_body(vin_ref, vp_ref, p0_ref, p1_ref, c0_ref, c1_ref,
                     nv_ref, ne_ref, nb_ref, w2_ref, b2_ref, w3_ref, b3_ref,
                     vout_ref, vsum_ref):
    j = pl.program_id(0)
    cnt = c0_ref[0][:, 0:1] + c1_ref[0][:, 0:1]
    esum = p0_ref[0] + p1_ref[0]
    emean = esum / jnp.maximum(cnt, 1.0)
    h = _dot(vp_ref[...], nv_ref[...]) + _dot(emean, ne_ref[...]) + nb_ref[...]
    h = _sp2(h)
    h = _sp2(_dot(h, w2_ref[...]) + b2_ref[...])
    vn = _sp2(_dot(h, w3_ref[...]) + b3_ref[...])
    vout_ref[...] = vn + vin_ref[...]

    @pl.when(j == 0)
    def _():
        vsum_ref[...] = jnp.zeros_like(vsum_ref)

    vsum_ref[...] += jnp.sum(vn, axis=0, keepdims=True)


def _node_block(v_in, vp, psum, cnt, f_w, nbias):
    (w1, _b1), (w2, b2), (w3, b3) = f_w
    nv, ne = w1[0:32], w1[32:64]
    return pl.pallas_call(
        _node_block_body,
        grid=(N_NODES // BN,),
        in_specs=[
            pl.BlockSpec((BN, D), lambda i: (i, 0)),
            pl.BlockSpec((BN, D), lambda i: (i, 0)),
            pl.BlockSpec((1, BN, D), lambda i: (0, i, 0)),
            pl.BlockSpec((1, BN, D), lambda i: (1, i, 0)),
            pl.BlockSpec((1, BN, D), lambda i: (0, i, 0)),
            pl.BlockSpec((1, BN, D), lambda i: (1, i, 0)),
            _full(nv.shape), _full(ne.shape), _full(nbias.shape),
            _full(w2.shape), _full((1, b2.shape[0])),
            _full(w3.shape), _full((1, b3.shape[0])),
        ],
        out_specs=[
            pl.BlockSpec((BN, D), lambda i: (i, 0)),
            pl.BlockSpec((1, D), lambda i: (0, 0)),
        ],
        out_shape=[
            jax.ShapeDtypeStruct((N_NODES, D), _F32),
            jax.ShapeDtypeStruct((1, D), _F32),
        ],
    )(v_in, vp, psum, psum, cnt, cnt, nv, ne, nbias,
      w2, b2.reshape(1, -1), w3, b3.reshape(1, -1))


def _s2s_body(feat_ref, h_ref, s_ref, r_ref, m_ref):
    j = pl.program_id(0)

    @pl.when(j == 0)
    def _():
        m_ref[0, 0] = -1e30
        s_ref[...] = jnp.zeros_like(s_ref)
        r_ref[...] = jnp.zeros_like(r_ref)

    feat = feat_ref[...]
    z = jnp.sum(feat * h_ref[...], axis=1, keepdims=True)
    m_old = m_ref[0, 0]
    m_new = jnp.maximum(m_old, jnp.max(z))
    scale = jnp.exp(m_old - m_new)
    p = jnp.exp(z - m_new)
    s_ref[...] = s_ref[...] * scale + jnp.sum(p).reshape(1, 1)
    r_ref[...] = r_ref[...] * scale + jnp.sum(p * feat, axis=0, keepdims=True)
    m_ref[0, 0] = m_new


def _s2s_pass(feat, h, rows, blk):
    return pl.pallas_call(
        _s2s_body,
        grid=(rows // blk,),
        in_specs=[
            pl.BlockSpec((blk, D), lambda i: (i, 0)),
            _full((1, D)),
        ],
        out_specs=[
            pl.BlockSpec((1, 1), lambda i: (0, 0)),
            pl.BlockSpec((1, D), lambda i: (0, 0)),
        ],
        out_shape=[
            jax.ShapeDtypeStruct((1, 1), _F32),
            jax.ShapeDtypeStruct((1, D), _F32),
        ],
        scratch_shapes=[pltpu.SMEM((1, 1), _F32)],
    )(feat, h)


def _sp2j(x):
    return jax.nn.softplus(x) - jnp.log(2.0)


def _mlp_j(layers, x, activate_last=True):
    n = len(layers)
    for i, (w, b) in enumerate(layers):
        x = x @ w + b
        if i < n - 1 or activate_last:
            x = _sp2j(x)
    return x


def _lstm_step(p, x, h, c):
    z = x @ p["W_ih"] + h @ p["W_hh"] + p["b"]
    i, f, g, o = jnp.split(z, 4, axis=-1)
    c = jax.nn.sigmoid(f) * c + jax.nn.sigmoid(i) * jnp.tanh(g)
    h = jax.nn.sigmoid(o) * jnp.tanh(c)
    return h, c


def _set2set(p, feat, rows, blk):
    h = jnp.zeros((1, D), _F32)
    c = jnp.zeros((1, D), _F32)
    q_star = jnp.zeros((1, 2 * D), _F32)
    for _ in range(2):
        h, c = _lstm_step(p, q_star, h, c)
        s, r_num = _s2s_pass(feat, h, rows, blk)
        r = r_num / s
        q_star = jnp.concatenate([h, r], axis=-1)
    return q_star


def kernel(edge_index, edge_feat, node_feat, state_feat, params):
    src = edge_index[0].astype(jnp.int32)
    dst = edge_index[1].astype(jnp.int32)
    p = params

    zrows = jnp.zeros((NPS, D), _F32)
    orows = jnp.ones((CH, D), _F32)

    u = _mlp_j(p["state_enc"], state_feat)
    v = _node_encode(node_feat, p["node_table"], p["node_enc"])
    e = None
    cnt = None

    for bi, bp in enumerate(p["blocks"]):
        has_pre = bool(bp["pre_e"])
        if has_pre:
            vp = _pre_node(v, bp["pre_n"])
            up = _mlp_j(p["blocks"][bi]["pre_s"], u)
            pre_e = bp["pre_e"]
            x = e
        else:
            vp, up = v, u
            pre_e = p["edge_enc"]
            x = edge_feat

        vs, vd = _sc_gather2(vp, src, dst)

        fw = bp["edge_f"]
        ubias = (fw[0][1] + up @ fw[0][0][96:128]).reshape(1, -1)
        e_new, e_next, esum = _edge_block(x, vs, vd, pre_e, fw, ubias, has_pre)

        if bi == 0:
            psum, cnt = _sc_scatter(e_new, dst, zrows, orows, True)
        else:
            (psum,) = _sc_scatter(e_new, dst, zrows, orows, False)

        nw = bp["node_f"]
        nbias = (nw[0][1] + up @ nw[0][0][64:96]).reshape(1, -1)
        v_next, vsum = _node_block(v, vp, psum, cnt, nw, nbias)

        u_new = _mlp_j(bp["state_f"],
                       jnp.concatenate([esum / N_EDGES, vsum / N_NODES, up],
                                       axis=-1))
        e, v, u = e_next, v_next, u_new + u

    nvec = _set2set(p["node_s2s"], v, N_NODES, BN)
    evec = _set2set(p["edge_s2s"], e, N_EDGES, BE)
    vec = jnp.concatenate([nvec[0], evec[0], u[0]], axis=-1)
    out = _mlp_j(p["out"], vec, activate_last=False)
    return jnp.squeeze(out)


# speedup vs baseline: 2.0793x; 1.2615x over previous
import functools

import jax
import jax.numpy as jnp
from jax import lax
from jax.experimental import pallas as pl
from jax.experimental.pallas import tpu as pltpu
from jax.experimental.pallas import tpu_sc as plsc

N_NODES = 10000
N_EDGES = 320000
D = 32

NC, NS = 2, 16
NW = NC * NS
EPW = N_EDGES // NW
CH = 80
NCHUNK = EPW // CH
NPS = N_NODES // NS

BE = 2560
BN = 2000

_F32 = jnp.float32


def _sp2(x):
    return jnp.maximum(x, 0.0) + jnp.log1p(jnp.exp(-jnp.abs(x))) - 0.6931471805599453


def _dot(a, b):
    return jnp.dot(a, b, preferred_element_type=_F32)


def _sc_mesh():
    return plsc.VectorSubcoreMesh(core_axis_name="c", subcore_axis_name="s",
                                  num_cores=NC, num_subcores=NS)


_SC_PARAMS = pltpu.CompilerParams(use_tc_tiling_on_sc=False)


NB = 2000
NBCH = NB // CH
NBAT = EPW // NB
IPW = EPW // CH


def _sc_gather2(vp, src2, dst2):
    @functools.partial(
        pl.kernel,
        out_type=(jax.ShapeDtypeStruct((N_EDGES, D), _F32),
                  jax.ShapeDtypeStruct((N_EDGES, D), _F32)),
        mesh=_sc_mesh(),
        compiler_params=_SC_PARAMS,
        scratch_types=[
            pltpu.VMEM((IPW, CH), jnp.int32),
            pltpu.VMEM((NB, D), _F32),
            pltpu.SemaphoreType.DMA,
        ],
    )
    def k(vp_hbm, src_hbm, dst_hbm, os_hbm, od_hbm, idx_v, rows_v, sem):
        wid = lax.axis_index("s") * NC + lax.axis_index("c")
        base = wid * EPW
        ibase = wid * IPW

        for ih, oh in ((src_hbm, os_hbm), (dst_hbm, od_hbm)):
            pltpu.sync_copy(ih.at[pl.ds(ibase, IPW)], idx_v)

            def batch(b, carry):
                def fire(j, c2):
                    pltpu.async_copy(vp_hbm.at[idx_v.at[b * NBCH + j]],
                                     rows_v.at[pl.ds(j * CH, CH)], sem)
                    return c2

                lax.fori_loop(0, NBCH, fire, 0, unroll=False)
                pltpu.make_async_copy(vp_hbm.at[pl.ds(0, NB)], rows_v,
                                      sem).wait()
                pltpu.sync_copy(rows_v, oh.at[pl.ds(base + b * NB, NB)])
                return carry

            lax.fori_loop(0, NBAT, batch, 0, unroll=False)

    return k(vp, src2, dst2)


def _sc_scatter(e_new, dst2, zrows, orows, with_counts):
    outs = [jax.ShapeDtypeStruct((NC, N_NODES, D), _F32)]
    scratch = [
        pltpu.VMEM_SHARED((N_NODES, D), _F32),
        pltpu.VMEM((IPW, CH), jnp.int32),
        pltpu.VMEM((NB, D), _F32),
        pltpu.SemaphoreType.DMA,
    ]
    if with_counts:
        outs.append(jax.ShapeDtypeStruct((NC, N_NODES, D), _F32))
        scratch += [pltpu.VMEM_SHARED((N_NODES, D), _F32),
                    pltpu.VMEM((CH, D), _F32),
                    pltpu.SemaphoreType.DMA]

    @functools.partial(pl.kernel, out_type=tuple(outs), mesh=_sc_mesh(),
                       compiler_params=_SC_PARAMS, scratch_types=scratch)
    def k(enew_hbm, dst_hbm, z_hbm, o_hbm, *rest):
        if with_counts:
            (psum_hbm, cnt_hbm, shared, idx_v, rows_v, sem,
             shared_cnt, ones_v, csem) = rest
        else:
            (psum_hbm, shared, idx_v, rows_v, sem) = rest
        cid = lax.axis_index("c")
        sid = lax.axis_index("s")
        wid = sid * NC + cid
        base = wid * EPW
        ibase = wid * IPW
        pltpu.sync_copy(dst_hbm.at[pl.ds(ibase, IPW)], idx_v)
        pltpu.sync_copy(z_hbm, shared.at[pl.ds(sid * NPS, NPS)])
        if with_counts:
            pltpu.sync_copy(z_hbm, shared_cnt.at[pl.ds(sid * NPS, NPS)])
            pltpu.sync_copy(o_hbm, ones_v)
        plsc.subcore_barrier()

        def batch(b, carry):
            pltpu.sync_copy(enew_hbm.at[pl.ds(base + b * NB, NB)], rows_v)

            def fire(j, c2):
                row = idx_v.at[b * NBCH + j]
                pltpu.async_copy(rows_v.at[pl.ds(j * CH, CH)],
                                 shared.at[row], sem, add=True)
                if with_counts:
                    pltpu.async_copy(ones_v, shared_cnt.at[row], csem,
                                     add=True)
                return c2

            lax.fori_loop(0, NBCH, fire, 0, unroll=False)
            pltpu.make_async_copy(enew_hbm.at[pl.ds(0, NB)], rows_v,
                                  sem).wait()
            if with_counts:
                pltpu.make_async_copy(enew_hbm.at[pl.ds(0, NB)], rows_v,
                                      csem).wait()
            return carry

        lax.fori_loop(0, NBAT, batch, 0, unroll=False)
        plsc.subcore_barrier()
        pltpu.sync_copy(shared.at[pl.ds(sid * NPS, NPS)],
                        psum_hbm.at[cid, pl.ds(sid * NPS, NPS)])
        if with_counts:
            pltpu.sync_copy(shared_cnt.at[pl.ds(sid * NPS, NPS)],
                            cnt_hbm.at[cid, pl.ds(sid * NPS, NPS)])

    return k(e_new, dst2, zrows, orows)


def _full(shape):
    return pl.BlockSpec(shape, lambda i: (0,) * len(shape))


def _edge_kernel_body(has_pre, *refs):
    (x_ref, vs_ref, vd_ref, q1_ref, r1_ref, q2_ref, r2_ref,
     a_ref, b_ref, c_ref, ub_ref, w2_ref, b2_ref, w3_ref, b3_ref,
     enew_ref, eout_ref, esum_ref) = refs
    j = pl.program_id(0)

    x = x_ref[...]
    ep = _sp2(_dot(x, q1_ref[...]) + r1_ref[...])
    ep = _sp2(_dot(ep, q2_ref[...]) + r2_ref[...])
    e_res = ep if not has_pre else x

    h = (_dot(vs_ref[...], a_ref[...]) + _dot(vd_ref[...], b_ref[...])
         + _dot(ep, c_ref[...]) + ub_ref[...])
    h = _sp2(h)
    h = _sp2(_dot(h, w2_ref[...]) + b2_ref[...])
    en = _sp2(_dot(h, w3_ref[...]) + b3_ref[...])

    enew_ref[...] = en
    eout_ref[...] = en + e_res

    @pl.when(j == 0)
    def _():
        esum_ref[...] = jnp.zeros_like(esum_ref)

    esum_ref[...] += jnp.sum(en, axis=0, keepdims=True)


def _edge_block(x, vs, vd, pre_w, f_w, ubias, has_pre):
    xin = x.shape[1]
    (q1, r1), (q2, r2) = pre_w
    (w1, _b1), (w2, b2), (w3, b3) = f_w
    a, b, c = w1[0:32], w1[32:64], w1[64:96]
    grid = (N_EDGES // BE,)
    body = functools.partial(_edge_kernel_body, has_pre)
    return pl.pallas_call(
        body,
        grid=grid,
        in_specs=[
            pl.BlockSpec((BE, xin), lambda i: (i, 0)),
            pl.BlockSpec((BE, D), lambda i: (i, 0)),
            pl.BlockSpec((BE, D), lambda i: (i, 0)),
            _full(q1.shape), _full((1, r1.shape[0])),
            _full(q2.shape), _full((1, r2.shape[0])),
            _full(a.shape), _full(b.shape), _full(c.shape), _full(ubias.shape),
            _full(w2.shape), _full((1, b2.shape[0])),
            _full(w3.shape), _full((1, b3.shape[0])),
        ],
        out_specs=[
            pl.BlockSpec((BE, D), lambda i: (i, 0)),
            pl.BlockSpec((BE, D), lambda i: (i, 0)),
            pl.BlockSpec((1, D), lambda i: (0, 0)),
        ],
        out_shape=[
            jax.ShapeDtypeStruct((N_EDGES, D), _F32),
            jax.ShapeDtypeStruct((N_EDGES, D), _F32),
            jax.ShapeDtypeStruct((1, D), _F32),
        ],
    )(x, vs, vd, q1, r1.reshape(1, -1), q2, r2.reshape(1, -1),
      a, b, c, ubias, w2, b2.reshape(1, -1), w3, b3.reshape(1, -1))


def _node_encode_body(idx_ref, tab_ref, q1_ref, r1_ref, q2_ref, r2_ref, out_ref):
    idx = idx_ref[...]
    iota = lax.broadcasted_iota(jnp.int32, (BN, 96), 1)
    oh = (iota == idx).astype(_F32)
    v = _dot(oh, tab_ref[...])
    v = _sp2(_dot(v, q1_ref[...]) + r1_ref[...])
    out_ref[...] = _sp2(_dot(v, q2_ref[...]) + r2_ref[...])


def _node_encode(node_feat, table, enc_w):
    tab = jnp.pad(table, ((0, 96 - table.shape[0]), (0, 0)))
    idx2 = node_feat.reshape(N_NODES, 1).astype(jnp.int32)
    (q1, r1), (q2, r2) = enc_w
    return pl.pallas_call(
        _node_encode_body,
        grid=(N_NODES // BN,),
        in_specs=[
            pl.BlockSpec((BN, 1), lambda i: (i, 0)),
            _full(tab.shape),
            _full(q1.shape), _full((1, r1.shape[0])),
            _full(q2.shape), _full((1, r2.shape[0])),
        ],
        out_specs=pl.BlockSpec((BN, D), lambda i: (i, 0)),
        out_shape=jax.ShapeDtypeStruct((N_NODES, D), _F32),
    )(idx2, tab, q1, r1.reshape(1, -1), q2, r2.reshape(1, -1))


def _mlp2_body(x_ref, q1_ref, r1_ref, q2_ref, r2_ref, out_ref):
    x = x_ref[...]
    x = _sp2(_dot(x, q1_ref[...]) + r1_ref[...])
    out_ref[...] = _sp2(_dot(x, q2_ref[...]) + r2_ref[...])


def _pre_node(v, pre_w):
    (q1, r1), (q2, r2) = pre_w
    return pl.pallas_call(
        _mlp2_body,
        grid=(N_NODES // BN,),
        in_specs=[
            pl.BlockSpec((BN, D), lambda i: (i, 0)),
            _full(q1.shape), _full((1, r1.shape[0])),
            _full(q2.shape), _full((1, r2.shape[0])),
        ],
        out_specs=pl.BlockSpec((BN, D), lambda i: (i, 0)),
        out_shape=jax.ShapeDtypeStruct((N_NODES, D), _F32),
    )(v, q1, r1.reshape(1, -1), q2, r2.reshape(1, -1))


def _node_block_body(vin_ref, vp_ref, p0_ref, p1_ref, c0_ref, c1_ref,
                     nv_ref, ne_ref, nb_ref, w2_ref, b2_ref, w3_ref, b3_ref,
                     vout_ref, vsum_ref):
    j = pl.program_id(0)
    cnt = c0_ref[0][:, 0:1] + c1_ref[0][:, 0:1]
    esum = p0_ref[0] + p1_ref[0]
    emean = esum / jnp.maximum(cnt, 1.0)
    h = _dot(vp_ref[...], nv_ref[...]) + _dot(emean, ne_ref[...]) + nb_ref[...]
    h = _sp2(h)
    h = _sp2(_dot(h, w2_ref[...]) + b2_ref[...])
    vn = _sp2(_dot(h, w3_ref[...]) + b3_ref[...])
    vout_ref[...] = vn + vin_ref[...]

    @pl.when(j == 0)
    def _():
        vsum_ref[...] = jnp.zeros_like(vsum_ref)

    vsum_ref[...] += jnp.sum(vn, axis=0, keepdims=True)


def _node_block(v_in, vp, psum, cnt, f_w, nbias):
    (w1, _b1), (w2, b2), (w3, b3) = f_w
    nv, ne = w1[0:32], w1[32:64]
    return pl.pallas_call(
        _node_block_body,
        grid=(N_NODES // BN,),
        in_specs=[
            pl.BlockSpec((BN, D), lambda i: (i, 0)),
            pl.BlockSpec((BN, D), lambda i: (i, 0)),
            pl.BlockSpec((1, BN, D), lambda i: (0, i, 0)),
            pl.BlockSpec((1, BN, D), lambda i: (1, i, 0)),
            pl.BlockSpec((1, BN, D), lambda i: (0, i, 0)),
            pl.BlockSpec((1, BN, D), lambda i: (1, i, 0)),
            _full(nv.shape), _full(ne.shape), _full(nbias.shape),
            _full(w2.shape), _full((1, b2.shape[0])),
            _full(w3.shape), _full((1, b3.shape[0])),
        ],
        out_specs=[
            pl.BlockSpec((BN, D), lambda i: (i, 0)),
            pl.BlockSpec((1, D), lambda i: (0, 0)),
        ],
        out_shape=[
            jax.ShapeDtypeStruct((N_NODES, D), _F32),
            jax.ShapeDtypeStruct((1, D), _F32),
        ],
    )(v_in, vp, psum, psum, cnt, cnt, nv, ne, nbias,
      w2, b2.reshape(1, -1), w3, b3.reshape(1, -1))


def _s2s_body(feat_ref, h_ref, s_ref, r_ref, m_ref):
    j = pl.program_id(0)

    @pl.when(j == 0)
    def _():
        m_ref[0, 0] = -1e30
        s_ref[...] = jnp.zeros_like(s_ref)
        r_ref[...] = jnp.zeros_like(r_ref)

    feat = feat_ref[...]
    z = jnp.sum(feat * h_ref[...], axis=1, keepdims=True)
    m_old = m_ref[0, 0]
    m_new = jnp.maximum(m_old, jnp.max(z))
    scale = jnp.exp(m_old - m_new)
    p = jnp.exp(z - m_new)
    s_ref[...] = s_ref[...] * scale + jnp.sum(p).reshape(1, 1)
    r_ref[...] = r_ref[...] * scale + jnp.sum(p * feat, axis=0, keepdims=True)
    m_ref[0, 0] = m_new


def _s2s_pass(feat, h, rows, blk):
    return pl.pallas_call(
        _s2s_body,
        grid=(rows // blk,),
        in_specs=[
            pl.BlockSpec((blk, D), lambda i: (i, 0)),
            _full((1, D)),
        ],
        out_specs=[
            pl.BlockSpec((1, 1), lambda i: (0, 0)),
            pl.BlockSpec((1, D), lambda i: (0, 0)),
        ],
        out_shape=[
            jax.ShapeDtypeStruct((1, 1), _F32),
            jax.ShapeDtypeStruct((1, D), _F32),
        ],
        scratch_shapes=[pltpu.SMEM((1, 1), _F32)],
    )(feat, h)


def _sp2j(x):
    return jax.nn.softplus(x) - jnp.log(2.0)


def _mlp_j(layers, x, activate_last=True):
    n = len(layers)
    for i, (w, b) in enumerate(layers):
        x = x @ w + b
        if i < n - 1 or activate_last:
            x = _sp2j(x)
    return x


def _lstm_step(p, x, h, c):
    z = x @ p["W_ih"] + h @ p["W_hh"] + p["b"]
    i, f, g, o = jnp.split(z, 4, axis=-1)
    c = jax.nn.sigmoid(f) * c + jax.nn.sigmoid(i) * jnp.tanh(g)
    h = jax.nn.sigmoid(o) * jnp.tanh(c)
    return h, c


def _set2set(p, feat, rows, blk):
    h = jnp.zeros((1, D), _F32)
    c = jnp.zeros((1, D), _F32)
    q_star = jnp.zeros((1, 2 * D), _F32)
    for _ in range(2):
        h, c = _lstm_step(p, q_star, h, c)
        s, r_num = _s2s_pass(feat, h, rows, blk)
        r = r_num / s
        q_star = jnp.concatenate([h, r], axis=-1)
    return q_star


def kernel(edge_index, edge_feat, node_feat, state_feat, params):
    src2 = edge_index[0].astype(jnp.int32).reshape(N_EDGES // CH, CH)
    dst2 = edge_index[1].astype(jnp.int32).reshape(N_EDGES // CH, CH)
    p = params

    zrows = jnp.zeros((NPS, D), _F32)
    orows = jnp.ones((CH, D), _F32)

    u = _mlp_j(p["state_enc"], state_feat)
    v = _node_encode(node_feat, p["node_table"], p["node_enc"])
    e = None
    cnt = None

    for bi, bp in enumerate(p["blocks"]):
        has_pre = bool(bp["pre_e"])
        if has_pre:
            vp = _pre_node(v, bp["pre_n"])
            up = _mlp_j(p["blocks"][bi]["pre_s"], u)
            pre_e = bp["pre_e"]
            x = e
        else:
            vp, up = v, u
            pre_e = p["edge_enc"]
            x = edge_feat

        vs, vd = _sc_gather2(vp, src2, dst2)

        fw = bp["edge_f"]
        ubias = (fw[0][1] + up @ fw[0][0][96:128]).reshape(1, -1)
        e_new, e_next, esum = _edge_block(x, vs, vd, pre_e, fw, ubias, has_pre)

        if bi == 0:
            psum, cnt = _sc_scatter(e_new, dst2, zrows, orows, True)
        else:
            (psum,) = _sc_scatter(e_new, dst2, zrows, orows, False)

        nw = bp["node_f"]
        nbias = (nw[0][1] + up @ nw[0][0][64:96]).reshape(1, -1)
        v_next, vsum = _node_block(v, vp, psum, cnt, nw, nbias)

        u_new = _mlp_j(bp["state_f"],
                       jnp.concatenate([esum / N_EDGES, vsum / N_NODES, up],
                                       axis=-1))
        e, v, u = e_next, v_next, u_new + u

    nvec = _set2set(p["node_s2s"], v, N_NODES, BN)
    evec = _set2set(p["edge_s2s"], e, N_EDGES, BE)
    vec = jnp.concatenate([nvec[0], evec[0], u[0]], axis=-1)
    out = _mlp_j(p["out"], vec, activate_last=False)
    return jnp.squeeze(out)


# speedup vs baseline: 2.5910x; 1.2461x over previous
import functools

import jax
import jax.numpy as jnp
from jax import lax
from jax.experimental import pallas as pl
from jax.experimental.pallas import tpu as pltpu
from jax.experimental.pallas import tpu_sc as plsc

N_NODES = 10000
N_EDGES = 320000
D = 32

NC, NS = 2, 16
NW = NC * NS
EPW = N_EDGES // NW
CH = 80
NCHUNK = EPW // CH
NPS = N_NODES // NS

BE = 6400
BN = 2000

_F32 = jnp.float32


_LOG2E = 1.4426950408889634
_LN2 = 0.6931471805599453


def _sp2_raw(x):
    t = jnp.minimum(x * _LOG2E, 126.0)
    return jnp.log2(1.0 + jnp.exp2(t))


def _sp2(x):
    return _LN2 * (_sp2_raw(x) - 1.0)


def _fold(w, b):
    return _LN2 * w, b - _LN2 * jnp.sum(w, axis=0)


def _dot(a, b):
    return jnp.dot(a, b, preferred_element_type=_F32)


def _sc_mesh():
    return plsc.VectorSubcoreMesh(core_axis_name="c", subcore_axis_name="s",
                                  num_cores=NC, num_subcores=NS)


_SC_PARAMS = pltpu.CompilerParams(use_tc_tiling_on_sc=False)


NB = 2000
NBCH = NB // CH
NBAT = EPW // NB
IPW = EPW // CH


def _sc_gather2(vp, src2, dst2):
    @functools.partial(
        pl.kernel,
        out_type=(jax.ShapeDtypeStruct((N_EDGES, D), _F32),
                  jax.ShapeDtypeStruct((N_EDGES, D), _F32)),
        mesh=_sc_mesh(),
        compiler_params=_SC_PARAMS,
        scratch_types=[
            pltpu.VMEM((IPW, CH), jnp.int32),
            pltpu.VMEM((NB, D), _F32),
            pltpu.SemaphoreType.DMA,
        ],
    )
    def k(vp_hbm, src_hbm, dst_hbm, os_hbm, od_hbm, idx_v, rows_v, sem):
        wid = lax.axis_index("s") * NC + lax.axis_index("c")
        base = wid * EPW
        ibase = wid * IPW

        for ih, oh in ((src_hbm, os_hbm), (dst_hbm, od_hbm)):
            pltpu.sync_copy(ih.at[pl.ds(ibase, IPW)], idx_v)

            def batch(b, carry):
                def fire(j, c2):
                    pltpu.async_copy(vp_hbm.at[idx_v.at[b * NBCH + j]],
                                     rows_v.at[pl.ds(j * CH, CH)], sem)
                    return c2

                lax.fori_loop(0, NBCH, fire, 0, unroll=False)
                pltpu.make_async_copy(vp_hbm.at[pl.ds(0, NB)], rows_v,
                                      sem).wait()
                pltpu.sync_copy(rows_v, oh.at[pl.ds(base + b * NB, NB)])
                return carry

            lax.fori_loop(0, NBAT, batch, 0, unroll=False)

    return k(vp, src2, dst2)


def _sc_scatter(e_new, dst2, zrows, orows, with_counts):
    outs = [jax.ShapeDtypeStruct((NC, N_NODES, D), _F32)]
    scratch = [
        pltpu.VMEM_SHARED((N_NODES, D), _F32),
        pltpu.VMEM((IPW, CH), jnp.int32),
        pltpu.VMEM((NB, D), _F32),
        pltpu.SemaphoreType.DMA,
    ]
    if with_counts:
        outs.append(jax.ShapeDtypeStruct((NC, N_NODES, D), _F32))
        scratch += [pltpu.VMEM_SHARED((N_NODES, D), _F32),
                    pltpu.VMEM((CH, D), _F32),
                    pltpu.SemaphoreType.DMA]

    @functools.partial(pl.kernel, out_type=tuple(outs), mesh=_sc_mesh(),
                       compiler_params=_SC_PARAMS, scratch_types=scratch)
    def k(enew_hbm, dst_hbm, z_hbm, o_hbm, *rest):
        if with_counts:
            (psum_hbm, cnt_hbm, shared, idx_v, rows_v, sem,
             shared_cnt, ones_v, csem) = rest
        else:
            (psum_hbm, shared, idx_v, rows_v, sem) = rest
        cid = lax.axis_index("c")
        sid = lax.axis_index("s")
        wid = sid * NC + cid
        base = wid * EPW
        ibase = wid * IPW
        pltpu.sync_copy(dst_hbm.at[pl.ds(ibase, IPW)], idx_v)
        pltpu.sync_copy(z_hbm, shared.at[pl.ds(sid * NPS, NPS)])
        if with_counts:
            pltpu.sync_copy(z_hbm, shared_cnt.at[pl.ds(sid * NPS, NPS)])
            pltpu.sync_copy(o_hbm, ones_v)
        plsc.subcore_barrier()

        def batch(b, carry):
            pltpu.sync_copy(enew_hbm.at[pl.ds(base + b * NB, NB)], rows_v)

            def fire(j, c2):
                row = idx_v.at[b * NBCH + j]
                pltpu.async_copy(rows_v.at[pl.ds(j * CH, CH)],
                                 shared.at[row], sem, add=True)
                if with_counts:
                    pltpu.async_copy(ones_v, shared_cnt.at[row], csem,
                                     add=True)
                return c2

            lax.fori_loop(0, NBCH, fire, 0, unroll=False)
            pltpu.make_async_copy(enew_hbm.at[pl.ds(0, NB)], rows_v,
                                  sem).wait()
            if with_counts:
                pltpu.make_async_copy(enew_hbm.at[pl.ds(0, NB)], rows_v,
                                      csem).wait()
            return carry

        lax.fori_loop(0, NBAT, batch, 0, unroll=False)
        plsc.subcore_barrier()
        pltpu.sync_copy(shared.at[pl.ds(sid * NPS, NPS)],
                        psum_hbm.at[cid, pl.ds(sid * NPS, NPS)])
        if with_counts:
            pltpu.sync_copy(shared_cnt.at[pl.ds(sid * NPS, NPS)],
                            cnt_hbm.at[cid, pl.ds(sid * NPS, NPS)])

    return k(e_new, dst2, zrows, orows)


def _full(shape):
    return pl.BlockSpec(shape, lambda i: (0,) * len(shape))


def _edge_kernel_body(has_pre, *refs):
    (x_ref, vs_ref, vd_ref, q1_ref, r1_ref, q2_ref, r2_ref,
     a_ref, b_ref, c_ref, ub_ref, w2_ref, b2_ref, w3_ref, b3_ref,
     enew_ref, eout_ref, esum_ref) = refs
    j = pl.program_id(0)

    x = x_ref[...]
    h1 = _sp2_raw(_dot(x, q1_ref[...]) + r1_ref[...])
    if has_pre:
        ep = _sp2_raw(_dot(h1, q2_ref[...]) + r2_ref[...])
        e_res = x
    else:
        ep = _sp2(_dot(h1, q2_ref[...]) + r2_ref[...])
        e_res = ep

    h = (_dot(vs_ref[...], a_ref[...]) + _dot(vd_ref[...], b_ref[...])
         + _dot(ep, c_ref[...]) + ub_ref[...])
    h = _sp2_raw(h)
    h = _sp2_raw(_dot(h, w2_ref[...]) + b2_ref[...])
    en = _sp2(_dot(h, w3_ref[...]) + b3_ref[...])

    enew_ref[...] = en
    eout_ref[...] = en + e_res

    @pl.when(j == 0)
    def _():
        esum_ref[...] = jnp.zeros_like(esum_ref)

    esum_ref[...] += jnp.sum(en, axis=0, keepdims=True)


def _edge_block(x, vs, vd, pre_w, f_w, up, has_pre):
    xin = x.shape[1]
    (q1, r1), (q2, r2) = pre_w
    q2, r2 = _fold(q2, r2)
    (w1, b1), (w2, b2), (w3, b3) = f_w
    a, b, c = w1[0:32], w1[32:64], w1[64:96]
    ubias = (b1 + up @ w1[96:128]).reshape(1, -1)
    if has_pre:
        ubias = ubias - _LN2 * jnp.sum(a + b + c, axis=0)
        a, b, c = _LN2 * a, _LN2 * b, _LN2 * c
    w2, b2 = _fold(w2, b2)
    w3, b3 = _fold(w3, b3)
    grid = (N_EDGES // BE,)
    body = functools.partial(_edge_kernel_body, has_pre)
    return pl.pallas_call(
        body,
        grid=grid,
        in_specs=[
            pl.BlockSpec((BE, xin), lambda i: (i, 0)),
            pl.BlockSpec((BE, D), lambda i: (i, 0)),
            pl.BlockSpec((BE, D), lambda i: (i, 0)),
            _full(q1.shape), _full((1, r1.shape[0])),
            _full(q2.shape), _full((1, r2.shape[0])),
            _full(a.shape), _full(b.shape), _full(c.shape), _full(ubias.shape),
            _full(w2.shape), _full((1, b2.shape[0])),
            _full(w3.shape), _full((1, b3.shape[0])),
        ],
        out_specs=[
            pl.BlockSpec((BE, D), lambda i: (i, 0)),
            pl.BlockSpec((BE, D), lambda i: (i, 0)),
            pl.BlockSpec((1, D), lambda i: (0, 0)),
        ],
        out_shape=[
            jax.ShapeDtypeStruct((N_EDGES, D), _F32),
            jax.ShapeDtypeStruct((N_EDGES, D), _F32),
            jax.ShapeDtypeStruct((1, D), _F32),
        ],
    )(x, vs, vd, q1, r1.reshape(1, -1), q2, r2.reshape(1, -1),
      a, b, c, ubias, w2, b2.reshape(1, -1), w3, b3.reshape(1, -1))


def _node_encode_body(idx_ref, tab_ref, q1_ref, r1_ref, q2_ref, r2_ref, out_ref):
    idx = idx_ref[...]
    iota = lax.broadcasted_iota(jnp.int32, (BN, 96), 1)
    oh = (iota == idx).astype(_F32)
    v = _dot(oh, tab_ref[...])
    v = _sp2_raw(_dot(v, q1_ref[...]) + r1_ref[...])
    out_ref[...] = _sp2(_dot(v, q2_ref[...]) + r2_ref[...])


def _node_encode(node_feat, table, enc_w):
    tab = jnp.pad(table, ((0, 96 - table.shape[0]), (0, 0)))
    idx2 = node_feat.reshape(N_NODES, 1).astype(jnp.int32)
    (q1, r1), (q2, r2) = enc_w
    q2, r2 = _fold(q2, r2)
    return pl.pallas_call(
        _node_encode_body,
        grid=(N_NODES // BN,),
        in_specs=[
            pl.BlockSpec((BN, 1), lambda i: (i, 0)),
            _full(tab.shape),
            _full(q1.shape), _full((1, r1.shape[0])),
            _full(q2.shape), _full((1, r2.shape[0])),
        ],
        out_specs=pl.BlockSpec((BN, D), lambda i: (i, 0)),
        out_shape=jax.ShapeDtypeStruct((N_NODES, D), _F32),
    )(idx2, tab, q1, r1.reshape(1, -1), q2, r2.reshape(1, -1))


def _mlp2_body(x_ref, q1_ref, r1_ref, q2_ref, r2_ref, out_ref):
    x = x_ref[...]
    x = _sp2_raw(_dot(x, q1_ref[...]) + r1_ref[...])
    out_ref[...] = _sp2_raw(_dot(x, q2_ref[...]) + r2_ref[...])


def _pre_node(v, pre_w):
    (q1, r1), (q2, r2) = pre_w
    q2, r2 = _fold(q2, r2)
    return pl.pallas_call(
        _mlp2_body,
        grid=(N_NODES // BN,),
        in_specs=[
            pl.BlockSpec((BN, D), lambda i: (i, 0)),
            _full(q1.shape), _full((1, r1.shape[0])),
            _full(q2.shape), _full((1, r2.shape[0])),
        ],
        out_specs=pl.BlockSpec((BN, D), lambda i: (i, 0)),
        out_shape=jax.ShapeDtypeStruct((N_NODES, D), _F32),
    )(v, q1, r1.reshape(1, -1), q2, r2.reshape(1, -1))


def _node_block_body(vin_ref, vp_ref, p0_ref, p1_ref, c0_ref, c1_ref,
                     nv_ref, ne_ref, nb_ref, w2_ref, b2_ref, w3_ref, b3_ref,
                     vout_ref, vsum_ref):
    j = pl.program_id(0)
    cnt = c0_ref[0][:, 0:1] + c1_ref[0][:, 0:1]
    esum = p0_ref[0] + p1_ref[0]
    emean = esum / jnp.maximum(cnt, 1.0)
    h = _dot(vp_ref[...], nv_ref[...]) + _dot(emean, ne_ref[...]) + nb_ref[...]
    h = _sp2_raw(h)
    h = _sp2_raw(_dot(h, w2_ref[...]) + b2_ref[...])
    vn = _sp2(_dot(h, w3_ref[...]) + b3_ref[...])
    vout_ref[...] = vn + vin_ref[...]

    @pl.when(j == 0)
    def _():
        vsum_ref[...] = jnp.zeros_like(vsum_ref)

    vsum_ref[...] += jnp.sum(vn, axis=0, keepdims=True)


def _node_block(v_in, vp, psum, cnt, f_w, up, vp_raw):
    (w1, b1), (w2, b2), (w3, b3) = f_w
    nv, ne = w1[0:32], w1[32:64]
    nbias = (b1 + up @ w1[64:96]).reshape(1, -1)
    if vp_raw:
        nbias = nbias - _LN2 * jnp.sum(nv, axis=0)
        nv = _LN2 * nv
    w2, b2 = _fold(w2, b2)
    w3, b3 = _fold(w3, b3)
    return pl.pallas_call(
        _node_block_body,
        grid=(N_NODES // BN,),
        in_specs=[
            pl.BlockSpec((BN, D), lambda i: (i, 0)),
            pl.BlockSpec((BN, D), lambda i: (i, 0)),
            pl.BlockSpec((1, BN, D), lambda i: (0, i, 0)),
            pl.BlockSpec((1, BN, D), lambda i: (1, i, 0)),
            pl.BlockSpec((1, BN, D), lambda i: (0, i, 0)),
            pl.BlockSpec((1, BN, D), lambda i: (1, i, 0)),
            _full(nv.shape), _full(ne.shape), _full(nbias.shape),
            _full(w2.shape), _full((1, b2.shape[0])),
            _full(w3.shape), _full((1, b3.shape[0])),
        ],
        out_specs=[
            pl.BlockSpec((BN, D), lambda i: (i, 0)),
            pl.BlockSpec((1, D), lambda i: (0, 0)),
        ],
        out_shape=[
            jax.ShapeDtypeStruct((N_NODES, D), _F32),
            jax.ShapeDtypeStruct((1, D), _F32),
        ],
    )(v_in, vp, psum, psum, cnt, cnt, nv, ne, nbias,
      w2, b2.reshape(1, -1), w3, b3.reshape(1, -1))


def _s2s_body(feat_ref, h_ref, s_ref, r_ref, m_ref):
    j = pl.program_id(0)

    @pl.when(j == 0)
    def _():
        m_ref[0, 0] = -1e30
        s_ref[...] = jnp.zeros_like(s_ref)
        r_ref[...] = jnp.zeros_like(r_ref)

    feat = feat_ref[...]
    z = jnp.sum(feat * h_ref[...], axis=1, keepdims=True)
    m_old = m_ref[0, 0]
    m_new = jnp.maximum(m_old, jnp.max(z))
    scale = jnp.exp2(m_old - m_new)
    p = jnp.exp2(z - m_new)
    s_ref[...] = s_ref[...] * scale + jnp.sum(p).reshape(1, 1)
    r_ref[...] = r_ref[...] * scale + jnp.sum(p * feat, axis=0, keepdims=True)
    m_ref[0, 0] = m_new


def _s2s_pass(feat, h, rows, blk):
    return pl.pallas_call(
        _s2s_body,
        grid=(rows // blk,),
        in_specs=[
            pl.BlockSpec((blk, D), lambda i: (i, 0)),
            _full((1, D)),
        ],
        out_specs=[
            pl.BlockSpec((1, 1), lambda i: (0, 0)),
            pl.BlockSpec((1, D), lambda i: (0, 0)),
        ],
        out_shape=[
            jax.ShapeDtypeStruct((1, 1), _F32),
            jax.ShapeDtypeStruct((1, D), _F32),
        ],
        scratch_shapes=[pltpu.SMEM((1, 1), _F32)],
    )(feat, h)


def _sp2j(x):
    return jax.nn.softplus(x) - jnp.log(2.0)


def _mlp_j(layers, x, activate_last=True):
    n = len(layers)
    for i, (w, b) in enumerate(layers):
        x = x @ w + b
        if i < n - 1 or activate_last:
            x = _sp2j(x)
    return x


def _lstm_step(p, x, h, c):
    z = x @ p["W_ih"] + h @ p["W_hh"] + p["b"]
    i, f, g, o = jnp.split(z, 4, axis=-1)
    c = jax.nn.sigmoid(f) * c + jax.nn.sigmoid(i) * jnp.tanh(g)
    h = jax.nn.sigmoid(o) * jnp.tanh(c)
    return h, c


def _set2set(p, feat, rows, blk):
    h = jnp.zeros((1, D), _F32)
    c = jnp.zeros((1, D), _F32)
    q_star = jnp.zeros((1, 2 * D), _F32)
    for _ in range(2):
        h, c = _lstm_step(p, q_star, h, c)
        s, r_num = _s2s_pass(feat, h * _LOG2E, rows, blk)
        r = r_num / s
        q_star = jnp.concatenate([h, r], axis=-1)
    return q_star


def kernel(edge_index, edge_feat, node_feat, state_feat, params):
    src2 = edge_index[0].astype(jnp.int32).reshape(N_EDGES // CH, CH)
    dst2 = edge_index[1].astype(jnp.int32).reshape(N_EDGES // CH, CH)
    p = params

    zrows = jnp.zeros((NPS, D), _F32)
    orows = jnp.ones((CH, D), _F32)

    u = _mlp_j(p["state_enc"], state_feat)
    v = _node_encode(node_feat, p["node_table"], p["node_enc"])
    e = None
    cnt = None

    for bi, bp in enumerate(p["blocks"]):
        has_pre = bool(bp["pre_e"])
        if has_pre:
            vp = _pre_node(v, bp["pre_n"])
            up = _mlp_j(p["blocks"][bi]["pre_s"], u)
            pre_e = bp["pre_e"]
            x = e
        else:
            vp, up = v, u
            pre_e = p["edge_enc"]
            x = edge_feat

        vs, vd = _sc_gather2(vp, src2, dst2)

        e_new, e_next, esum = _edge_block(x, vs, vd, pre_e, bp["edge_f"], up,
                                          has_pre)

        if bi == 0:
            psum, cnt = _sc_scatter(e_new, dst2, zrows, orows, True)
        else:
            (psum,) = _sc_scatter(e_new, dst2, zrows, orows, False)

        v_next, vsum = _node_block(v, vp, psum, cnt, bp["node_f"], up, has_pre)

        u_new = _mlp_j(bp["state_f"],
                       jnp.concatenate([esum / N_EDGES, vsum / N_NODES, up],
                                       axis=-1))
        e, v, u = e_next, v_next, u_new + u

    nvec = _set2set(p["node_s2s"], v, N_NODES, BN)
    evec = _set2set(p["edge_s2s"], e, N_EDGES, BE)
    vec = jnp.concatenate([nvec[0], evec[0], u[0]], axis=-1)
    out = _mlp_j(p["out"], vec, activate_last=False)
    return jnp.squeeze(out)
